# baseline (speedup 1.0000x reference)
import functools

import jax
import jax.numpy as jnp
from jax import lax
from jax.experimental import pallas as pl
from jax.experimental.pallas import tpu as pltpu
from jax.experimental.pallas import tpu_sc as plsc

BATCH = 4096
SEQ = 200
D = 64
TABP = 1024

_info = plsc.get_sparse_core_info()
_NC, _NS = _info.num_cores, _info.num_subcores
NW = _NC * _NS
BBLK = BATCH // NW

_mesh = plsc.VectorSubcoreMesh(core_axis_name="c", subcore_axis_name="s")


@functools.partial(
    pl.kernel,
    out_type=jax.ShapeDtypeStruct((SEQ, 8, NW, 8, 128), jnp.float32),
    mesh=_mesh,
    scratch_types=[
        pltpu.VMEM((SEQ, BBLK), jnp.int32),
        pltpu.VMEM((D, TABP), jnp.float32),
        pltpu.VMEM((D, BBLK), jnp.float32),
        pltpu.VMEM((D, BBLK), jnp.float32),
        pltpu.SemaphoreType.DMA,
        pltpu.SemaphoreType.DMA,
    ],
    compiler_params=pltpu.CompilerParams(
        use_tc_tiling_on_sc=False, needs_layout_passes=False
    ),
)
def _embed(idx_hbm, table_hbm, out_hbm, idx_v, tab_v, obuf0, obuf1, sw0, sw1):
    wid = lax.axis_index("s") * _NC + lax.axis_index("c")
    obuf = (obuf0, obuf1)
    sw = (sw0, sw1)

    pltpu.sync_copy(idx_hbm.at[:, pl.ds(wid * BBLK, BBLK)], idx_v)
    pltpu.sync_copy(table_hbm, tab_v)

    def compute(s, b):
        ob = obuf[b]
        idxs = [idx_v[s, pl.ds(16 * k, 16)] for k in range(BBLK // 16)]

        @plsc.parallel_loop(0, D, unroll=8)
        def _d(d):
            dfull = jnp.full((16,), d, jnp.int32)
            for k in range(BBLK // 16):
                v = plsc.load_gather(tab_v, [dfull, idxs[k]])
                ob[d, pl.ds(16 * k, 16)] = v

    def wb_start(s, b):
        for dt in range(8):
            pltpu.async_copy(
                obuf[b].at[pl.ds(8 * dt, 8), :], out_hbm.at[s, dt, wid], sw[b]
            )

    def wb_wait(s, b):
        for dt in range(8):
            pltpu.make_async_copy(
                obuf[b].at[pl.ds(8 * dt, 8), :], out_hbm.at[s, dt, wid], sw[b]
            ).wait()

    compute(0, 0)
    wb_start(0, 0)
    compute(1, 1)
    wb_start(1, 1)

    def body(g, carry):
        for b in range(2):
            s = 2 * g + b
            wb_wait(s - 2, b)
            compute(s, b)
            wb_start(s, b)
        return carry

    lax.fori_loop(1, SEQ // 2, body, 0)

    wb_wait(SEQ - 2, 0)
    wb_wait(SEQ - 1, 1)


def kernel(word_sequences, embedding_weight):
    idx_t = word_sequences.T
    tab_t = jnp.pad(embedding_weight.T, ((0, 0), (0, TABP - embedding_weight.shape[0])))
    out5 = _embed(idx_t, tab_t)
    return out5.transpose(2, 4, 0, 1, 3).reshape(BATCH, SEQ, D)


# speedup vs baseline: 1.0282x; 1.0282x over previous
import functools

import jax
import jax.numpy as jnp
from jax import lax
from jax.experimental import pallas as pl
from jax.experimental.pallas import tpu as pltpu
from jax.experimental.pallas import tpu_sc as plsc

BATCH = 4096
SEQ = 200
D = 64
TABP = 1024

_info = plsc.get_sparse_core_info()
_NC, _NS = _info.num_cores, _info.num_subcores
NW = _NC * _NS
BBLK = BATCH // NW

_mesh = plsc.VectorSubcoreMesh(core_axis_name="c", subcore_axis_name="s")


@functools.partial(
    pl.kernel,
    out_type=jax.ShapeDtypeStruct((SEQ, 8, NW, 8, 128), jnp.float32),
    mesh=_mesh,
    scratch_types=[
        pltpu.VMEM((SEQ, BBLK), jnp.int32),
        pltpu.VMEM((D, TABP), jnp.float32),
        pltpu.VMEM((8, 8, BBLK), jnp.float32),
        pltpu.VMEM((8, 8, BBLK), jnp.float32),
        pltpu.SemaphoreType.DMA,
        pltpu.SemaphoreType.DMA,
    ],
    compiler_params=pltpu.CompilerParams(
        use_tc_tiling_on_sc=False, needs_layout_passes=False
    ),
)
def _embed(idx_hbm, table_hbm, out_hbm, idx_v, tab_v, obuf0, obuf1, sw0, sw1):
    wid = lax.axis_index("s") * _NC + lax.axis_index("c")
    obuf = (obuf0, obuf1)
    sw = (sw0, sw1)

    pltpu.sync_copy(idx_hbm.at[:, pl.ds(wid * BBLK, BBLK)], idx_v)
    pltpu.sync_copy(table_hbm, tab_v)

    def compute(s, b):
        ob = obuf[b]
        idxs = [idx_v[s, pl.ds(16 * k, 16)] for k in range(BBLK // 16)]

        @plsc.parallel_loop(0, D, unroll=4)
        def _d(d):
            dfull = jnp.full((16,), d, jnp.int32)
            for k in range(BBLK // 16):
                v = plsc.load_gather(tab_v, [dfull, idxs[k]])
                ob[d // 8, d % 8, pl.ds(16 * k, 16)] = v

    def wb_start(s, b):
        pltpu.async_copy(obuf[b], out_hbm.at[s, :, wid], sw[b])

    def wb_wait(s, b):
        pltpu.make_async_copy(obuf[b], out_hbm.at[s, :, wid], sw[b]).wait()

    compute(0, 0)
    wb_start(0, 0)
    compute(1, 1)
    wb_start(1, 1)

    def body(g, carry):
        for b in range(2):
            s = 2 * g + b
            wb_wait(s - 2, b)
            compute(s, b)
            wb_start(s, b)
        return carry

    lax.fori_loop(1, SEQ // 2, body, 0)

    wb_wait(SEQ - 2, 0)
    wb_wait(SEQ - 1, 1)


def kernel(word_sequences, embedding_weight):
    idx_t = word_sequences.T
    tab_t = jnp.pad(embedding_weight.T, ((0, 0), (0, TABP - embedding_weight.shape[0])))
    out5 = _embed(idx_t, tab_t)
    return out5.transpose(2, 4, 0, 1, 3).reshape(BATCH, SEQ, D)


# speedup vs baseline: 1.0723x; 1.0429x over previous
import functools

import jax
import jax.numpy as jnp
from jax import lax
from jax.experimental import pallas as pl
from jax.experimental.pallas import tpu as pltpu
from jax.experimental.pallas import tpu_sc as plsc

BATCH = 4096
SEQ = 200
D = 64
TABP = 1024

_info = plsc.get_sparse_core_info()
_NC, _NS = _info.num_cores, _info.num_subcores
NW = _NC * _NS
BBLK = BATCH // NW

_mesh = plsc.VectorSubcoreMesh(core_axis_name="c", subcore_axis_name="s")


@functools.partial(
    pl.kernel,
    out_type=jax.ShapeDtypeStruct((SEQ, 8, NW, 8, 128), jnp.float32),
    mesh=_mesh,
    scratch_types=[
        pltpu.VMEM((SEQ, BBLK), jnp.int32),
        pltpu.VMEM((D, TABP), jnp.float32),
        pltpu.VMEM((8, 8, BBLK), jnp.float32),
        pltpu.VMEM((8, 8, BBLK), jnp.float32),
        pltpu.VMEM((8, 8, BBLK), jnp.float32),
        pltpu.VMEM((8, 8, BBLK), jnp.float32),
        pltpu.SemaphoreType.DMA,
        pltpu.SemaphoreType.DMA,
        pltpu.SemaphoreType.DMA,
        pltpu.SemaphoreType.DMA,
    ],
    compiler_params=pltpu.CompilerParams(
        use_tc_tiling_on_sc=False, needs_layout_passes=False
    ),
)
def _embed(
    idx_hbm, table_hbm, out_hbm, idx_v, tab_v, ob0, ob1, ob2, ob3, sw0, sw1, sw2, sw3
):
    wid = lax.axis_index("s") * _NC + lax.axis_index("c")
    obuf = (ob0, ob1, ob2, ob3)
    sw = (sw0, sw1, sw2, sw3)
    NBUF = 4

    pltpu.sync_copy(idx_hbm.at[:, pl.ds(wid * BBLK, BBLK)], idx_v)
    pltpu.sync_copy(table_hbm, tab_v)

    def compute(s, b):
        ob = obuf[b]
        idxs = [idx_v[s, pl.ds(16 * k, 16)] for k in range(BBLK // 16)]

        @plsc.parallel_loop(0, D, unroll=4)
        def _d(d):
            dfull = jnp.full((16,), d, jnp.int32)
            for k in range(BBLK // 16):
                v = plsc.load_gather(tab_v, [dfull, idxs[k]])
                ob[d // 8, d % 8, pl.ds(16 * k, 16)] = v

    def wb_start(s, b):
        pltpu.async_copy(obuf[b], out_hbm.at[s, :, wid], sw[b])

    def wb_wait(s, b):
        pltpu.make_async_copy(obuf[b], out_hbm.at[s, :, wid], sw[b]).wait()

    for b in range(NBUF):
        compute(b, b)
        wb_start(b, b)

    def body(g, carry):
        for b in range(NBUF):
            s = NBUF * g + b
            wb_wait(s - NBUF, b)
            compute(s, b)
            wb_start(s, b)
        return carry

    lax.fori_loop(1, SEQ // NBUF, body, 0)

    for b in range(NBUF):
        wb_wait(SEQ - NBUF + b, b)


def kernel(word_sequences, embedding_weight):
    idx_t = word_sequences.T
    tab_t = jnp.pad(embedding_weight.T, ((0, 0), (0, TABP - embedding_weight.shape[0])))
    out5 = _embed(idx_t, tab_t)
    return out5.transpose(2, 4, 0, 1, 3).reshape(BATCH, SEQ, D)
